# baseline (speedup 1.0000x reference)
import functools

import jax
import jax.numpy as jnp
from jax import lax
from jax.experimental import pallas as pl
from jax.experimental.pallas import tpu as pltpu
from jax.experimental.pallas import tpu_sc as plsc

N = 4
C = 64
L = 8192
H = 4
CR = 16
CHUNK = 8
NBINS = 256
FLAT = H * L
TS = 512
NTS = L // TS
TA = 256
NTA = L // TA
EPS = 1e-5

_i32 = jnp.int32
_f32 = jnp.float32


def _conv_body(xt_ref, w0_ref, w1_ref, w2_ref, wy_ref, xpre_ref, ypre_ref,
               stats_ref):
    b = pl.program_id(0)
    x = xt_ref[0]
    zrow = jnp.zeros((1, C), _f32)
    xm = jnp.concatenate([zrow, x[:-1, :]], axis=0)
    xp = jnp.concatenate([x[1:, :], zrow], axis=0)
    dn = (((1,), (0,)), ((), ()))
    xpre = (lax.dot_general(xm, w0_ref[...], dn)
            + lax.dot_general(x, w1_ref[...], dn)
            + lax.dot_general(xp, w2_ref[...], dn))
    ypre = lax.dot_general(x, wy_ref[...], dn)
    xpre_ref[0] = xpre
    ypre_ref[0] = ypre

    sx = jnp.sum(xpre, axis=0, keepdims=True)
    sx2 = jnp.sum(xpre * xpre, axis=0, keepdims=True)
    sy = jnp.sum(ypre, axis=0, keepdims=True)
    sy2 = jnp.sum(ypre * ypre, axis=0, keepdims=True)
    padx = jnp.zeros((1, 128 - CR), _f32)
    pady = jnp.zeros((1, 128 - C), _f32)
    blk = jnp.concatenate([
        jnp.concatenate([sx, padx], axis=1),
        jnp.concatenate([sx2, padx], axis=1),
        jnp.concatenate([sy, pady], axis=1),
        jnp.concatenate([sy2, pady], axis=1),
        jnp.zeros((4, 128), _f32),
    ], axis=0)

    @pl.when(b == 0)
    def _():
        stats_ref[...] = jnp.zeros((8, 128), _f32)
    stats_ref[...] += blk


def _run_conv(xt, w0, w1, w2, wy):
    return pl.pallas_call(
        _conv_body,
        grid=(N,),
        in_specs=[
            pl.BlockSpec((1, L, C), lambda b: (b, 0, 0)),
            pl.BlockSpec((C, CR), lambda b: (0, 0)),
            pl.BlockSpec((C, CR), lambda b: (0, 0)),
            pl.BlockSpec((C, CR), lambda b: (0, 0)),
            pl.BlockSpec((C, C), lambda b: (0, 0)),
        ],
        out_specs=[
            pl.BlockSpec((1, L, CR), lambda b: (b, 0, 0)),
            pl.BlockSpec((1, L, C), lambda b: (b, 0, 0)),
            pl.BlockSpec((8, 128), lambda b: (0, 0)),
        ],
        out_shape=[
            jax.ShapeDtypeStruct((N, L, CR), _f32),
            jax.ShapeDtypeStruct((N, L, C), _f32),
            jax.ShapeDtypeStruct((8, 128), _f32),
        ],
    )(xt, w0, w1, w2, wy)


def _embed_body(xpre_ref, ypre_ref, params_ref, rot_ref, xy_ref, keys_ref):
    xs = params_ref[0:1, 0:CR]
    xb = params_ref[1:2, 0:CR]
    ys = params_ref[2:3, 0:C]
    yb = params_ref[3:4, 0:C]
    xe = jnp.maximum(xpre_ref[0] * xs + xb, 0.0)
    ye = jnp.maximum(ypre_ref[0] * ys + yb, 0.0)
    xy_ref[0] = jnp.concatenate(
        [xe, ye, jnp.zeros((L, 128 - CR - C), _f32)], axis=1)
    rot = lax.dot_general(xe, rot_ref[...], (((1,), (0,)), ((), ())))
    cols = []
    iota64 = lax.broadcasted_iota(_i32, (L, 64), 1)
    for h in range(H):
        rh = rot[:, h * 64:(h + 1) * 64]
        m = jnp.max(jnp.abs(rh), axis=1, keepdims=True)
        pos = jnp.min(jnp.where(rh == m, iota64, 128), axis=1, keepdims=True)
        neg = jnp.min(jnp.where(-rh == m, iota64, 128), axis=1, keepdims=True)
        code = jnp.where(pos < 128, pos, neg + 64)
        cols.append(code + h)
    keys_ref[0] = jnp.concatenate(cols, axis=1)


def _run_embed(xpre, ypre, params, rot2):
    return pl.pallas_call(
        _embed_body,
        grid=(N,),
        in_specs=[
            pl.BlockSpec((1, L, CR), lambda b: (b, 0, 0)),
            pl.BlockSpec((1, L, C), lambda b: (b, 0, 0)),
            pl.BlockSpec((8, 128), lambda b: (0, 0)),
            pl.BlockSpec((CR, H * 64), lambda b: (0, 0)),
        ],
        out_specs=[
            pl.BlockSpec((1, L, 128), lambda b: (b, 0, 0)),
            pl.BlockSpec((1, L, H), lambda b: (b, 0, 0)),
        ],
        out_shape=[
            jax.ShapeDtypeStruct((N, L, 128), _f32),
            jax.ShapeDtypeStruct((N, L, H), _i32),
        ],
    )(xpre, ypre, params, rot2)


def _keycol_onehot(keys_ref, h):
    kblk = keys_ref[0]
    hsel = lax.broadcasted_iota(_i32, (TS, H), 1) == h
    kcol = jnp.max(jnp.where(hsel, kblk, -1), axis=1, keepdims=True)
    return (kcol == lax.broadcasted_iota(_i32, (TS, NBINS), 1)).astype(_f32)


def _hist_body(keys_ref, hist_ref):
    h = pl.program_id(1)
    t = pl.program_id(2)
    onehot = _keycol_onehot(keys_ref, h)

    @pl.when(jnp.logical_and(h == 0, t == 0))
    def _():
        hist_ref[...] = jnp.zeros((1, 8, NBINS), _f32)
    hist_ref[0, 0:1, :] += jnp.sum(onehot, axis=0, keepdims=True)


def _run_hist(keys):
    return pl.pallas_call(
        _hist_body,
        grid=(N, H, NTS),
        in_specs=[pl.BlockSpec((1, TS, H), lambda b, h, t: (b, t, 0))],
        out_specs=pl.BlockSpec((1, 8, NBINS), lambda b, h, t: (b, 0, 0)),
        out_shape=jax.ShapeDtypeStruct((N, 8, NBINS), _f32),
    )(keys)


def _pos_body(keys_ref, hist_ref, pos_ref, scr_ref):
    b = pl.program_id(0)
    h = pl.program_id(1)
    t = pl.program_id(2)
    onehot = _keycol_onehot(keys_ref, h)

    @pl.when(jnp.logical_and(h == 0, t == 0))
    def _():
        scr_ref[0:1, :] = jnp.zeros((1, NBINS), _f32)

    hist = hist_ref[0, 0:1, :]
    lt = (lax.broadcasted_iota(_i32, (NBINS, NBINS), 0)
          < lax.broadcasted_iota(_i32, (NBINS, NBINS), 1)).astype(_f32)
    base = lax.dot_general(hist, lt, (((1,), (0,)), ((), ())),
                           precision=lax.Precision.HIGHEST)
    tri = (lax.broadcasted_iota(_i32, (TS, TS), 0)
           >= lax.broadcasted_iota(_i32, (TS, TS), 1)).astype(jnp.bfloat16)
    cum = lax.dot_general(tri, onehot.astype(jnp.bfloat16),
                          (((1,), (0,)), ((), ())),
                          preferred_element_type=_f32)
    vec = scr_ref[0:1, :] + base
    posf = jnp.sum((cum + vec) * onehot, axis=1, keepdims=True) - 1.0
    scr_ref[0:1, :] += jnp.sum(onehot, axis=0, keepdims=True)
    pos_ref[0] = posf.astype(_i32) + b * FLAT


def _run_sort(keys):
    hist = _run_hist(keys)
    nrow = N * H * NTS
    return pl.pallas_call(
        _pos_body,
        grid=(N, H, NTS),
        in_specs=[
            pl.BlockSpec((1, TS, H), lambda b, h, t: (b, t, 0)),
            pl.BlockSpec((1, 8, NBINS), lambda b, h, t: (b, 0, 0)),
        ],
        out_specs=[
            pl.BlockSpec((1, TS, 1),
                         lambda b, h, t: (b * (H * NTS) + h * NTS + t, 0, 0)),
        ],
        out_shape=[jax.ShapeDtypeStruct((nrow, TS, 1), _i32)],
        scratch_shapes=[pltpu.VMEM((8, NBINS), _f32)],
    )(keys, hist)


_NWORK = 32
_RPW = N * FLAT // _NWORK
_CH = 512


def _scatter_body(xy_hbm, idx_hbm, xys_hbm, idxv, xv, sem):
    cid = lax.axis_index("c")
    sid = lax.axis_index("s")
    wid = sid * 2 + cid
    for j in range(_RPW // 1024):
        r0 = pl.multiple_of(wid * _RPW + j * 1024, 1024)
        pltpu.sync_copy(idx_hbm.at[pl.ds(pl.multiple_of(r0 // 128, 8), 8)],
                        idxv)
        for k in range(2):
            rk = pl.multiple_of(r0 + k * _CH, _CH)
            src0 = pl.multiple_of((rk // FLAT) * L + lax.rem(rk, L), _CH)
            pltpu.sync_copy(xy_hbm.at[pl.ds(src0, _CH)], xv)
            descs = []
            for jj in range(4):
                row = idxv.at[k * 4 + jj]
                descs.append(pltpu.async_copy(
                    xv.at[pl.ds(jj * 128, 128)], xys_hbm.at[row], sem))
            for d in descs:
                d.wait()


@functools.lru_cache(maxsize=None)
def _sc_mesh():
    return plsc.VectorSubcoreMesh(core_axis_name="c", subcore_axis_name="s")


@functools.lru_cache(maxsize=None)
def _make_scatter_call():
    return pl.kernel(
        _scatter_body,
        out_type=[jax.ShapeDtypeStruct((N * FLAT, 128), _f32)],
        mesh=_sc_mesh(),
        scratch_types=[
            pltpu.VMEM((8, 128), _i32),
            pltpu.VMEM((_CH, 128), _f32),
            pltpu.SemaphoreType.DMA,
        ],
    )


def _scatter_call(xy, idx2d):
    return _make_scatter_call()(xy, idx2d)


def _att_body(xy_ref, mask_ref, att_ref, bs_ref):
    t = pl.program_id(2)
    start = t * TA
    pstart = lax.rem(start + L - CHUNK, L)
    nstart = lax.rem(start + TA, L)

    q = xy_ref[0, pl.ds(start, TA), 0:CR]
    xk = jnp.concatenate([
        xy_ref[0, pl.ds(pstart, CHUNK), 0:CR],
        xy_ref[0, pl.ds(start, TA), 0:CR],
        xy_ref[0, pl.ds(nstart, CHUNK), 0:CR],
    ], axis=0)
    nrm = jnp.sqrt(jnp.sum(xk * xk, axis=1, keepdims=True))
    xn = xk / jnp.maximum(nrm, 5e-5)

    s = lax.dot_general(q, xn, (((1,), (1,)), ((), ())))
    sm = s + mask_ref[...]
    m = jnp.max(sm, axis=1, keepdims=True)
    e = jnp.exp(sm - m)
    ssum = jnp.sum(e, axis=1, keepdims=True)
    bs_ref[0] = m + jnp.log(ssum)
    prob = e * (1.0 / ssum)

    yk = jnp.concatenate([
        xy_ref[0, pl.ds(pstart, CHUNK), CR:CR + C],
        xy_ref[0, pl.ds(start, TA), CR:CR + C],
        xy_ref[0, pl.ds(nstart, CHUNK), CR:CR + C],
    ], axis=0)
    att = lax.dot_general(prob, yk, (((1,), (0,)), ((), ())))
    att_ref[0] = jnp.concatenate(
        [att, jnp.zeros((TA, 128 - C), _f32)], axis=1)


def _run_att(xy3, mask):
    nrow = N * H * NTA
    return pl.pallas_call(
        _att_body,
        grid=(N, H, NTA),
        in_specs=[
            pl.BlockSpec((1, L, 128), lambda b, g, t: (b * H + g, 0, 0)),
            pl.BlockSpec((TA, TA + 2 * CHUNK), lambda b, g, t: (0, 0)),
        ],
        out_specs=[
            pl.BlockSpec((1, TA, 128),
                         lambda b, g, t: (b * (H * NTA) + g * NTA + t, 0, 0)),
            pl.BlockSpec((1, TA, 1),
                         lambda b, g, t: (b * (H * NTA) + g * NTA + t, 0, 0)),
        ],
        out_shape=[
            jax.ShapeDtypeStruct((nrow, TA, 128), _f32),
            jax.ShapeDtypeStruct((nrow, TA, 1), _f32),
        ],
    )(xy3, mask)


def _gather_body(att_hbm, idx_hbm, out_hbm, idxv, buf, sem):
    cid = lax.axis_index("c")
    sid = lax.axis_index("s")
    wid = sid * 2 + cid
    for j in range(_RPW // 1024):
        r0 = pl.multiple_of(wid * _RPW + j * 1024, 1024)
        pltpu.sync_copy(idx_hbm.at[pl.ds(pl.multiple_of(r0 // 128, 8), 8)],
                        idxv)
        for k in range(2):
            rk = pl.multiple_of(r0 + k * _CH, _CH)
            descs = []
            for jj in range(4):
                row = idxv.at[k * 4 + jj]
                descs.append(pltpu.async_copy(
                    att_hbm.at[row], buf.at[pl.ds(jj * 128, 128)], sem))
            for d in descs:
                d.wait()
            pltpu.sync_copy(buf, out_hbm.at[pl.ds(rk, _CH)])


@functools.lru_cache(maxsize=None)
def _make_gather_call():
    return pl.kernel(
        _gather_body,
        out_type=[jax.ShapeDtypeStruct((N * FLAT, 128), _f32)],
        mesh=_sc_mesh(),
        scratch_types=[
            pltpu.VMEM((8, 128), _i32),
            pltpu.VMEM((_CH, 128), _f32),
            pltpu.SemaphoreType.DMA,
        ],
    )


def _gather_call(att, idx2d):
    return _make_gather_call()(att, idx2d)


def _combine_body(att_ref, bs_ref, pre_ref, stats_ref):
    b = pl.program_id(0)
    t = pl.program_id(1)
    a = [att_ref[0, h, :, 0:C] for h in range(H)]
    s = [bs_ref[0, h] for h in range(H)]
    m = jnp.maximum(jnp.maximum(s[0], s[1]), jnp.maximum(s[2], s[3]))
    w = [jnp.exp(si - m) for si in s]
    wsum = w[0] + w[1] + w[2] + w[3]
    out = (a[0] * w[0] + a[1] * w[1] + a[2] * w[2] + a[3] * w[3]) / wsum
    pre_ref[0] = out

    sy = jnp.sum(out, axis=0, keepdims=True)
    sy2 = jnp.sum(out * out, axis=0, keepdims=True)
    pad = jnp.zeros((1, 128 - C), _f32)
    blk = jnp.concatenate([
        jnp.concatenate([sy, pad], axis=1),
        jnp.concatenate([sy2, pad], axis=1),
        jnp.zeros((6, 128), _f32),
    ], axis=0)

    @pl.when(jnp.logical_and(b == 0, t == 0))
    def _():
        stats_ref[...] = jnp.zeros((8, 128), _f32)
    stats_ref[...] += blk


def _run_combine(att_g, bs4):
    return pl.pallas_call(
        _combine_body,
        grid=(N, NTA),
        in_specs=[
            pl.BlockSpec((1, H, TA, 128), lambda b, t: (b, 0, t, 0)),
            pl.BlockSpec((1, H, TA, 1), lambda b, t: (b, 0, t, 0)),
        ],
        out_specs=[
            pl.BlockSpec((1, TA, C), lambda b, t: (b * NTA + t, 0, 0)),
            pl.BlockSpec((8, 128), lambda b, t: (0, 0)),
        ],
        out_shape=[
            jax.ShapeDtypeStruct((N * NTA, TA, C), _f32),
            jax.ShapeDtypeStruct((8, 128), _f32),
        ],
    )(att_g, bs4)


def _final_body(pre_ref, xt_ref, params_ref, out_ref):
    sc = params_ref[0:1, 0:C]
    bi = params_ref[1:2, 0:C]
    out_ref[0] = pre_ref[0] * sc + bi + xt_ref[0]


def _run_final(pre3, xt, params):
    return pl.pallas_call(
        _final_body,
        grid=(N,),
        in_specs=[
            pl.BlockSpec((1, L, C), lambda b: (b, 0, 0)),
            pl.BlockSpec((1, L, C), lambda b: (b, 0, 0)),
            pl.BlockSpec((8, 128), lambda b: (0, 0)),
        ],
        out_specs=pl.BlockSpec((1, L, C), lambda b: (b, 0, 0)),
        out_shape=jax.ShapeDtypeStruct((N, L, C), _f32),
    )(pre3, xt, params)


def _pack_params(xscale, xbias, yscale, ybias):
    p = jnp.zeros((8, 128), _f32)
    p = p.at[0, :xscale.shape[0]].set(xscale)
    p = p.at[1, :xbias.shape[0]].set(xbias)
    p = p.at[2, :yscale.shape[0]].set(yscale)
    p = p.at[3, :ybias.shape[0]].set(ybias)
    return p


def _affine(g, bparam, ssum, ssq, count):
    mean = ssum / count
    var = ssq / count - mean * mean
    scale = g / jnp.sqrt(var + EPS)
    return scale, bparam - mean * scale


@jax.jit
def kernel(input_tensor, conv_match_w, bn1_g, bn1_b, conv_asm_w, bn2_g,
           bn2_b, bn3_g, bn3_b, random_rotations):
    xt = jnp.transpose(input_tensor, (0, 2, 1))
    w0 = jnp.transpose(conv_match_w[:, :, 0], (1, 0))
    w1 = jnp.transpose(conv_match_w[:, :, 1], (1, 0))
    w2 = jnp.transpose(conv_match_w[:, :, 2], (1, 0))
    wy = jnp.transpose(conv_asm_w[:, :, 0], (1, 0))
    rot2 = jnp.transpose(random_rotations, (0, 1, 2)).reshape(CR, H * 64)

    xpre, ypre, st1 = _run_conv(xt, w0, w1, w2, wy)
    cnt = float(N * L)
    xs_, xb_ = _affine(bn1_g, bn1_b, st1[0, :CR], st1[1, :CR], cnt)
    ys_, yb_ = _affine(bn2_g, bn2_b, st1[2, :C], st1[3, :C], cnt)
    params1 = _pack_params(xs_, xb_, ys_, yb_)

    xy, keys = _run_embed(xpre, ypre, params1, rot2)

    pos, = _run_sort(keys)
    idx2d = pos.reshape(N * FLAT // 128, 128)

    xy_srt, = _scatter_call(xy.reshape(N * L, 128), idx2d)

    qc = jnp.arange(TA)[:, None] // CHUNK
    kc = jnp.arange(TA + 2 * CHUNK)[None, :] // CHUNK - 1
    mask = jnp.where(jnp.abs(qc - kc) <= 1, 0.0, -1e30).astype(_f32)
    att_s, bs_s = _run_att(xy_srt.reshape(N * H, L, 128), mask)

    att_g, = _gather_call(att_s.reshape(N * FLAT, 128), idx2d)

    pre, st3 = _run_combine(att_g.reshape(N, H, L, 128),
                            bs_s.reshape(N, H, L, 1))

    fs_, fb_ = _affine(bn3_g, bn3_b, st3[0, :C], st3[1, :C], cnt)
    params3 = _pack_params(fs_, fb_, fs_, fb_)
    out = _run_final(pre.reshape(N, L, C), xt, params3)
    return jnp.transpose(out, (0, 2, 1))


# speedup vs baseline: 1.0822x; 1.0822x over previous
import functools

import jax
import jax.numpy as jnp
from jax import lax
from jax.experimental import pallas as pl
from jax.experimental.pallas import tpu as pltpu
from jax.experimental.pallas import tpu_sc as plsc

N = 4
C = 64
L = 8192
H = 4
CR = 16
CHUNK = 8
NBINS = 256
FLAT = H * L
TS = 512
NTS = L // TS
TA = 512
NTA = L // TA
EPS = 1e-5

_i32 = jnp.int32
_f32 = jnp.float32


def _conv_body(xt_ref, w0_ref, w1_ref, w2_ref, wy_ref, xpre_ref, ypre_ref,
               stats_ref):
    b = pl.program_id(0)
    x = xt_ref[0]
    zrow = jnp.zeros((1, C), _f32)
    xm = jnp.concatenate([zrow, x[:-1, :]], axis=0)
    xp = jnp.concatenate([x[1:, :], zrow], axis=0)
    dn = (((1,), (0,)), ((), ()))
    xpre = (lax.dot_general(xm, w0_ref[...], dn)
            + lax.dot_general(x, w1_ref[...], dn)
            + lax.dot_general(xp, w2_ref[...], dn))
    ypre = lax.dot_general(x, wy_ref[...], dn)
    xpre_ref[0] = xpre
    ypre_ref[0] = ypre

    sx = jnp.sum(xpre, axis=0, keepdims=True)
    sx2 = jnp.sum(xpre * xpre, axis=0, keepdims=True)
    sy = jnp.sum(ypre, axis=0, keepdims=True)
    sy2 = jnp.sum(ypre * ypre, axis=0, keepdims=True)
    padx = jnp.zeros((1, 128 - CR), _f32)
    pady = jnp.zeros((1, 128 - C), _f32)
    blk = jnp.concatenate([
        jnp.concatenate([sx, padx], axis=1),
        jnp.concatenate([sx2, padx], axis=1),
        jnp.concatenate([sy, pady], axis=1),
        jnp.concatenate([sy2, pady], axis=1),
        jnp.zeros((4, 128), _f32),
    ], axis=0)

    @pl.when(b == 0)
    def _():
        stats_ref[...] = jnp.zeros((8, 128), _f32)
    stats_ref[...] += blk


def _run_conv(xt, w0, w1, w2, wy):
    return pl.pallas_call(
        _conv_body,
        grid=(N,),
        in_specs=[
            pl.BlockSpec((1, L, C), lambda b: (b, 0, 0)),
            pl.BlockSpec((C, CR), lambda b: (0, 0)),
            pl.BlockSpec((C, CR), lambda b: (0, 0)),
            pl.BlockSpec((C, CR), lambda b: (0, 0)),
            pl.BlockSpec((C, C), lambda b: (0, 0)),
        ],
        out_specs=[
            pl.BlockSpec((1, L, CR), lambda b: (b, 0, 0)),
            pl.BlockSpec((1, L, C), lambda b: (b, 0, 0)),
            pl.BlockSpec((8, 128), lambda b: (0, 0)),
        ],
        out_shape=[
            jax.ShapeDtypeStruct((N, L, CR), _f32),
            jax.ShapeDtypeStruct((N, L, C), _f32),
            jax.ShapeDtypeStruct((8, 128), _f32),
        ],
    )(xt, w0, w1, w2, wy)


def _embed_body(xpre_ref, ypre_ref, params_ref, rot_ref, xy_ref, keys_ref):
    xs = params_ref[0:1, 0:CR]
    xb = params_ref[1:2, 0:CR]
    ys = params_ref[2:3, 0:C]
    yb = params_ref[3:4, 0:C]
    xe = jnp.maximum(xpre_ref[0] * xs + xb, 0.0)
    ye = jnp.maximum(ypre_ref[0] * ys + yb, 0.0)
    xy_ref[0] = jnp.concatenate(
        [xe, ye, jnp.zeros((L, 128 - CR - C), _f32)], axis=1)
    rot = lax.dot_general(xe, rot_ref[...], (((1,), (0,)), ((), ())))
    cols = []
    iota64 = lax.broadcasted_iota(_i32, (L, 64), 1)
    for h in range(H):
        rh = rot[:, h * 64:(h + 1) * 64]
        m = jnp.max(jnp.abs(rh), axis=1, keepdims=True)
        pos = jnp.min(jnp.where(rh == m, iota64, 128), axis=1, keepdims=True)
        neg = jnp.min(jnp.where(-rh == m, iota64, 128), axis=1, keepdims=True)
        code = jnp.where(pos < 128, pos, neg + 64)
        cols.append(code + h)
    keys_ref[0] = jnp.concatenate(cols, axis=1)


def _run_embed(xpre, ypre, params, rot2):
    return pl.pallas_call(
        _embed_body,
        grid=(N,),
        in_specs=[
            pl.BlockSpec((1, L, CR), lambda b: (b, 0, 0)),
            pl.BlockSpec((1, L, C), lambda b: (b, 0, 0)),
            pl.BlockSpec((8, 128), lambda b: (0, 0)),
            pl.BlockSpec((CR, H * 64), lambda b: (0, 0)),
        ],
        out_specs=[
            pl.BlockSpec((1, L, 128), lambda b: (b, 0, 0)),
            pl.BlockSpec((1, L, H), lambda b: (b, 0, 0)),
        ],
        out_shape=[
            jax.ShapeDtypeStruct((N, L, 128), _f32),
            jax.ShapeDtypeStruct((N, L, H), _i32),
        ],
    )(xpre, ypre, params, rot2)


def _keycol_onehot(keys_ref, h):
    kblk = keys_ref[0]
    hsel = lax.broadcasted_iota(_i32, (TS, H), 1) == h
    kcol = jnp.max(jnp.where(hsel, kblk, -1), axis=1, keepdims=True)
    return (kcol == lax.broadcasted_iota(_i32, (TS, NBINS), 1)).astype(_f32)


def _hist_body(keys_ref, hist_ref):
    h = pl.program_id(1)
    t = pl.program_id(2)
    onehot = _keycol_onehot(keys_ref, h)

    @pl.when(jnp.logical_and(h == 0, t == 0))
    def _():
        hist_ref[...] = jnp.zeros((1, 8, NBINS), _f32)
    hist_ref[0, 0:1, :] += jnp.sum(onehot, axis=0, keepdims=True)


def _run_hist(keys):
    return pl.pallas_call(
        _hist_body,
        grid=(N, H, NTS),
        in_specs=[pl.BlockSpec((1, TS, H), lambda b, h, t: (b, t, 0))],
        out_specs=pl.BlockSpec((1, 8, NBINS), lambda b, h, t: (b, 0, 0)),
        out_shape=jax.ShapeDtypeStruct((N, 8, NBINS), _f32),
    )(keys)


def _pos_body(keys_ref, hist_ref, pos_ref, scr_ref):
    b = pl.program_id(0)
    h = pl.program_id(1)
    t = pl.program_id(2)
    onehot = _keycol_onehot(keys_ref, h)

    @pl.when(jnp.logical_and(h == 0, t == 0))
    def _():
        scr_ref[0:1, :] = jnp.zeros((1, NBINS), _f32)

    hist = hist_ref[0, 0:1, :]
    lt = (lax.broadcasted_iota(_i32, (NBINS, NBINS), 0)
          < lax.broadcasted_iota(_i32, (NBINS, NBINS), 1)).astype(_f32)
    base = lax.dot_general(hist, lt, (((1,), (0,)), ((), ())),
                           precision=lax.Precision.HIGHEST)
    tri = (lax.broadcasted_iota(_i32, (TS, TS), 0)
           >= lax.broadcasted_iota(_i32, (TS, TS), 1)).astype(jnp.bfloat16)
    cum = lax.dot_general(tri, onehot.astype(jnp.bfloat16),
                          (((1,), (0,)), ((), ())),
                          preferred_element_type=_f32)
    vec = scr_ref[0:1, :] + base
    posf = jnp.sum((cum + vec) * onehot, axis=1, keepdims=True) - 1.0
    scr_ref[0:1, :] += jnp.sum(onehot, axis=0, keepdims=True)
    pos_ref[0] = posf.astype(_i32) + b * FLAT


def _run_sort(keys):
    hist = _run_hist(keys)
    nrow = N * H * NTS
    return pl.pallas_call(
        _pos_body,
        grid=(N, H, NTS),
        in_specs=[
            pl.BlockSpec((1, TS, H), lambda b, h, t: (b, t, 0)),
            pl.BlockSpec((1, 8, NBINS), lambda b, h, t: (b, 0, 0)),
        ],
        out_specs=[
            pl.BlockSpec((1, TS, 1),
                         lambda b, h, t: (b * (H * NTS) + h * NTS + t, 0, 0)),
        ],
        out_shape=[jax.ShapeDtypeStruct((nrow, TS, 1), _i32)],
        scratch_shapes=[pltpu.VMEM((8, NBINS), _f32)],
    )(keys, hist)


_NWORK = 32
_RPW = N * FLAT // _NWORK
_CH = 512


def _scatter_body(xy_hbm, idx_hbm, xys_hbm, idxv, xv, sem):
    cid = lax.axis_index("c")
    sid = lax.axis_index("s")
    wid = sid * 2 + cid
    for j in range(_RPW // 1024):
        r0 = pl.multiple_of(wid * _RPW + j * 1024, 1024)
        pltpu.sync_copy(idx_hbm.at[pl.ds(pl.multiple_of(r0 // 128, 8), 8)],
                        idxv)
        for k in range(2):
            rk = pl.multiple_of(r0 + k * _CH, _CH)
            src0 = pl.multiple_of((rk // FLAT) * L + lax.rem(rk, L), _CH)
            pltpu.sync_copy(xy_hbm.at[pl.ds(src0, _CH)], xv)
            descs = []
            for jj in range(4):
                row = idxv.at[k * 4 + jj]
                descs.append(pltpu.async_copy(
                    xv.at[pl.ds(jj * 128, 128)], xys_hbm.at[row], sem))
            for d in descs:
                d.wait()


@functools.lru_cache(maxsize=None)
def _sc_mesh():
    return plsc.VectorSubcoreMesh(core_axis_name="c", subcore_axis_name="s")


@functools.lru_cache(maxsize=None)
def _make_scatter_call():
    return pl.kernel(
        _scatter_body,
        out_type=[jax.ShapeDtypeStruct((N * FLAT, 128), _f32)],
        mesh=_sc_mesh(),
        scratch_types=[
            pltpu.VMEM((8, 128), _i32),
            pltpu.VMEM((_CH, 128), _f32),
            pltpu.SemaphoreType.DMA,
        ],
    )


def _scatter_call(xy, idx2d):
    return _make_scatter_call()(xy, idx2d)


def _att_body(xy_ref, mask_ref, att_ref, bs_ref):
    t = pl.program_id(2)
    start = t * TA
    pstart = lax.rem(start + L - CHUNK, L)
    nstart = lax.rem(start + TA, L)

    q = xy_ref[0, pl.ds(start, TA), 0:CR]
    xk = jnp.concatenate([
        xy_ref[0, pl.ds(pstart, CHUNK), 0:CR],
        xy_ref[0, pl.ds(start, TA), 0:CR],
        xy_ref[0, pl.ds(nstart, CHUNK), 0:CR],
    ], axis=0)
    nrm = jnp.sqrt(jnp.sum(xk * xk, axis=1, keepdims=True))
    xn = xk / jnp.maximum(nrm, 5e-5)

    s = lax.dot_general(q, xn, (((1,), (1,)), ((), ())))
    sm = s + mask_ref[...]
    m = jnp.max(sm, axis=1, keepdims=True)
    e = jnp.exp(sm - m)
    ssum = jnp.sum(e, axis=1, keepdims=True)
    bs_ref[0] = m + jnp.log(ssum)
    prob = e * (1.0 / ssum)

    yk = jnp.concatenate([
        xy_ref[0, pl.ds(pstart, CHUNK), CR:CR + C],
        xy_ref[0, pl.ds(start, TA), CR:CR + C],
        xy_ref[0, pl.ds(nstart, CHUNK), CR:CR + C],
    ], axis=0)
    att = lax.dot_general(prob, yk, (((1,), (0,)), ((), ())))
    att_ref[0] = jnp.concatenate(
        [att, jnp.zeros((TA, 128 - C), _f32)], axis=1)


def _run_att(xy3, mask):
    nrow = N * H * NTA
    return pl.pallas_call(
        _att_body,
        grid=(N, H, NTA),
        in_specs=[
            pl.BlockSpec((1, L, 128), lambda b, g, t: (b * H + g, 0, 0)),
            pl.BlockSpec((TA, TA + 2 * CHUNK), lambda b, g, t: (0, 0)),
        ],
        out_specs=[
            pl.BlockSpec((1, TA, 128),
                         lambda b, g, t: (b * (H * NTA) + g * NTA + t, 0, 0)),
            pl.BlockSpec((1, TA, 1),
                         lambda b, g, t: (b * (H * NTA) + g * NTA + t, 0, 0)),
        ],
        out_shape=[
            jax.ShapeDtypeStruct((nrow, TA, 128), _f32),
            jax.ShapeDtypeStruct((nrow, TA, 1), _f32),
        ],
    )(xy3, mask)


def _gather_body(att_hbm, idx_hbm, out_hbm, idxv, buf, sem):
    cid = lax.axis_index("c")
    sid = lax.axis_index("s")
    wid = sid * 2 + cid
    for j in range(_RPW // 1024):
        r0 = pl.multiple_of(wid * _RPW + j * 1024, 1024)
        pltpu.sync_copy(idx_hbm.at[pl.ds(pl.multiple_of(r0 // 128, 8), 8)],
                        idxv)
        for k in range(2):
            rk = pl.multiple_of(r0 + k * _CH, _CH)
            descs = []
            for jj in range(4):
                row = idxv.at[k * 4 + jj]
                descs.append(pltpu.async_copy(
                    att_hbm.at[row], buf.at[pl.ds(jj * 128, 128)], sem))
            for d in descs:
                d.wait()
            pltpu.sync_copy(buf, out_hbm.at[pl.ds(rk, _CH)])


@functools.lru_cache(maxsize=None)
def _make_gather_call():
    return pl.kernel(
        _gather_body,
        out_type=[jax.ShapeDtypeStruct((N * FLAT, 128), _f32)],
        mesh=_sc_mesh(),
        scratch_types=[
            pltpu.VMEM((8, 128), _i32),
            pltpu.VMEM((_CH, 128), _f32),
            pltpu.SemaphoreType.DMA,
        ],
    )


def _gather_call(att, idx2d):
    return _make_gather_call()(att, idx2d)


def _combine_body(att_ref, bs_ref, pre_ref, stats_ref):
    b = pl.program_id(0)
    t = pl.program_id(1)
    a = [att_ref[0, h, :, 0:C] for h in range(H)]
    s = [bs_ref[0, h] for h in range(H)]
    m = jnp.maximum(jnp.maximum(s[0], s[1]), jnp.maximum(s[2], s[3]))
    w = [jnp.exp(si - m) for si in s]
    wsum = w[0] + w[1] + w[2] + w[3]
    out = (a[0] * w[0] + a[1] * w[1] + a[2] * w[2] + a[3] * w[3]) / wsum
    pre_ref[0] = out

    sy = jnp.sum(out, axis=0, keepdims=True)
    sy2 = jnp.sum(out * out, axis=0, keepdims=True)
    pad = jnp.zeros((1, 128 - C), _f32)
    blk = jnp.concatenate([
        jnp.concatenate([sy, pad], axis=1),
        jnp.concatenate([sy2, pad], axis=1),
        jnp.zeros((6, 128), _f32),
    ], axis=0)

    @pl.when(jnp.logical_and(b == 0, t == 0))
    def _():
        stats_ref[...] = jnp.zeros((8, 128), _f32)
    stats_ref[...] += blk


def _run_combine(att_g, bs4):
    return pl.pallas_call(
        _combine_body,
        grid=(N, NTA),
        in_specs=[
            pl.BlockSpec((1, H, TA, 128), lambda b, t: (b, 0, t, 0)),
            pl.BlockSpec((1, H, TA, 1), lambda b, t: (b, 0, t, 0)),
        ],
        out_specs=[
            pl.BlockSpec((1, TA, C), lambda b, t: (b * NTA + t, 0, 0)),
            pl.BlockSpec((8, 128), lambda b, t: (0, 0)),
        ],
        out_shape=[
            jax.ShapeDtypeStruct((N * NTA, TA, C), _f32),
            jax.ShapeDtypeStruct((8, 128), _f32),
        ],
    )(att_g, bs4)


def _final_body(pre_ref, xt_ref, params_ref, out_ref):
    sc = params_ref[0:1, 0:C]
    bi = params_ref[1:2, 0:C]
    out_ref[0] = pre_ref[0] * sc + bi + xt_ref[0]


def _run_final(pre3, xt, params):
    return pl.pallas_call(
        _final_body,
        grid=(N,),
        in_specs=[
            pl.BlockSpec((1, L, C), lambda b: (b, 0, 0)),
            pl.BlockSpec((1, L, C), lambda b: (b, 0, 0)),
            pl.BlockSpec((8, 128), lambda b: (0, 0)),
        ],
        out_specs=pl.BlockSpec((1, L, C), lambda b: (b, 0, 0)),
        out_shape=jax.ShapeDtypeStruct((N, L, C), _f32),
    )(pre3, xt, params)


def _pack_params(xscale, xbias, yscale, ybias):
    p = jnp.zeros((8, 128), _f32)
    p = p.at[0, :xscale.shape[0]].set(xscale)
    p = p.at[1, :xbias.shape[0]].set(xbias)
    p = p.at[2, :yscale.shape[0]].set(yscale)
    p = p.at[3, :ybias.shape[0]].set(ybias)
    return p


def _affine(g, bparam, ssum, ssq, count):
    mean = ssum / count
    var = ssq / count - mean * mean
    scale = g / jnp.sqrt(var + EPS)
    return scale, bparam - mean * scale


@jax.jit
def kernel(input_tensor, conv_match_w, bn1_g, bn1_b, conv_asm_w, bn2_g,
           bn2_b, bn3_g, bn3_b, random_rotations):
    xt = jnp.transpose(input_tensor, (0, 2, 1))
    w0 = jnp.transpose(conv_match_w[:, :, 0], (1, 0))
    w1 = jnp.transpose(conv_match_w[:, :, 1], (1, 0))
    w2 = jnp.transpose(conv_match_w[:, :, 2], (1, 0))
    wy = jnp.transpose(conv_asm_w[:, :, 0], (1, 0))
    rot2 = jnp.transpose(random_rotations, (0, 1, 2)).reshape(CR, H * 64)

    xpre, ypre, st1 = _run_conv(xt, w0, w1, w2, wy)
    cnt = float(N * L)
    xs_, xb_ = _affine(bn1_g, bn1_b, st1[0, :CR], st1[1, :CR], cnt)
    ys_, yb_ = _affine(bn2_g, bn2_b, st1[2, :C], st1[3, :C], cnt)
    params1 = _pack_params(xs_, xb_, ys_, yb_)

    xy, keys = _run_embed(xpre, ypre, params1, rot2)

    pos, = _run_sort(keys)
    idx2d = pos.reshape(N * FLAT // 128, 128)

    xy_srt, = _scatter_call(xy.reshape(N * L, 128), idx2d)

    qc = jnp.arange(TA)[:, None] // CHUNK
    kc = jnp.arange(TA + 2 * CHUNK)[None, :] // CHUNK - 1
    mask = jnp.where(jnp.abs(qc - kc) <= 1, 0.0, -1e30).astype(_f32)
    att_s, bs_s = _run_att(xy_srt.reshape(N * H, L, 128), mask)

    att_g, = _gather_call(att_s.reshape(N * FLAT, 128), idx2d)

    pre, st3 = _run_combine(att_g.reshape(N, H, L, 128),
                            bs_s.reshape(N, H, L, 1))

    fs_, fb_ = _affine(bn3_g, bn3_b, st3[0, :C], st3[1, :C], cnt)
    params3 = _pack_params(fs_, fb_, fs_, fb_)
    out = _run_final(pre.reshape(N, L, C), xt, params3)
    return jnp.transpose(out, (0, 2, 1))
